# Initial kernel scaffold; baseline (speedup 1.0000x reference)
import functools

import jax
import jax.numpy as jnp
from jax import lax
from jax.experimental import pallas as pl
from jax.experimental.pallas import tpu as pltpu
from jax.experimental.pallas import tpu_sc as plsc

N = 50000
E = 800000
EMB_DIM = 128
GIN_DIM = 64
CW = 32
NPAD = 53248
EPAD = 802816
XROWS = NPAD // 128
EROWS = EPAD // 128
TRASH = N
NC = 2
NS = 16
TPB = EROWS // NS
KB = 8
TROWS = NPAD // NS
BR = 1024
GRID = NPAD // BR

_mesh = plsc.VectorSubcoreMesh(core_axis_name="c", subcore_axis_name="s")


def _emb_gather(x2d, emb_chunks):
    H = len(emb_chunks)
    wpw = XROWS // (NC * NS)

    @functools.partial(
        pl.kernel,
        out_type=[jax.ShapeDtypeStruct((NPAD, CW), jnp.float32)] * H,
        mesh=_mesh,
        scratch_types=[
            pltpu.VMEM((1, 128), jnp.int32),
            pltpu.VMEM((H, 128, CW), jnp.float32),
            pltpu.SemaphoreType.DMA,
        ],
    )
    def k(x_ref, *rest):
        embs = rest[:H]
        outs = rest[H:2 * H]
        xv, rows, sem = rest[2 * H:]
        c = lax.axis_index("c")
        s = lax.axis_index("s")
        wid = s * NC + c

        def body(i, carry):
            rb = wid * wpw + i
            pltpu.sync_copy(x_ref.at[pl.ds(rb, 1)], xv)
            descs = [pltpu.async_copy(embs[h].at[xv.at[0]], rows.at[h], sem)
                     for h in range(H)]
            for d in descs:
                d.wait()
            for h in range(H):
                pltpu.sync_copy(rows.at[h], outs[h].at[pl.ds(rb * 128, 128)])
            return carry

        lax.fori_loop(0, wpw, body, 0)

    return list(k(x2d, *emb_chunks))


def _segment_sum(src2d, dst2d, zeros, node_chunks):
    H = len(node_chunks)

    @functools.partial(
        pl.kernel,
        out_type=[jax.ShapeDtypeStruct((NPAD, CW), jnp.float32)] * H,
        mesh=_mesh,
        scratch_types=[
            pltpu.VMEM((KB, 128), jnp.int32),
            pltpu.VMEM((KB, 128), jnp.int32),
            pltpu.VMEM((KB, 128, CW), jnp.float32),
            pltpu.VMEM_SHARED((NPAD, CW), jnp.float32),
            pltpu.SemaphoreType.DMA,
        ],
    )
    def k(src_ref, dst_ref, z_ref, *rest):
        nodes = rest[:H]
        outs = rest[H:2 * H]
        sv, dv, rows, acc, sem = rest[2 * H:]
        c = lax.axis_index("c")
        s = lax.axis_index("s")
        for h in range(H):
            @pl.when(c == (h % NC))
            def _(h=h):
                node_h = nodes[h]
                out_h = outs[h]
                pltpu.sync_copy(z_ref, acc.at[pl.ds(s * TROWS, TROWS)])
                plsc.subcore_barrier()

                def body(i, carry):
                    rb = s * TPB + i * KB
                    pltpu.sync_copy(src_ref.at[pl.ds(rb, KB)], sv)
                    pltpu.sync_copy(dst_ref.at[pl.ds(rb, KB)], dv)
                    descs = [pltpu.async_copy(node_h.at[sv.at[j]],
                                              rows.at[j], sem)
                             for j in range(KB)]
                    for d in descs:
                        d.wait()
                    for j in range(KB):
                        pltpu.sync_copy(rows.at[j], acc.at[dv.at[j]],
                                        add=True)
                    return carry

                lax.fori_loop(0, TPB // KB, body, 0)
                plsc.subcore_barrier()
                pltpu.sync_copy(acc.at[pl.ds(s * TROWS, TROWS)],
                                out_h.at[pl.ds(s * TROWS, TROWS)])

    return list(k(src2d, dst2d, zeros, *node_chunks))


def _mlp(node_chunks, agg_chunks, se, Wa, ba, g, be, Wb, bb):
    H = len(node_chunks)
    din = H * CW

    def body(*refs):
        nrefs = refs[:H]
        arefs = refs[H:2 * H]
        se_r, wa_r, ba_r, g_r, be_r, wb_r, bb_r = refs[2 * H:2 * H + 7]
        o0, o1, pool = refs[2 * H + 7:]
        i = pl.program_id(0)
        x = jnp.concatenate([r[...] for r in nrefs], axis=1)
        a = jnp.concatenate([r[...] for r in arefs], axis=1)
        z = se_r[...] * x + a
        h1 = jnp.dot(z, wa_r[...], preferred_element_type=jnp.float32) + ba_r[...]
        m = jnp.mean(h1, axis=1, keepdims=True)
        v = jnp.mean((h1 - m) ** 2, axis=1, keepdims=True)
        h1 = (h1 - m) * lax.rsqrt(v + 1e-5) * g_r[...] + be_r[...]
        h1 = jnp.maximum(h1, 0.0)
        y = jnp.dot(h1, wb_r[...], preferred_element_type=jnp.float32) + bb_r[...]
        o0[...] = y[:, :CW]
        o1[...] = y[:, CW:]
        rows = i * BR + lax.broadcasted_iota(jnp.int32, (BR, 1), 0)
        part = jnp.sum(jnp.where(rows < N, y, 0.0), axis=0, keepdims=True)
        pb = jnp.broadcast_to(part, (8, GIN_DIM))

        @pl.when(i == 0)
        def _init():
            pool[...] = pb

        @pl.when(i != 0)
        def _acc():
            pool[...] += pb

    cspec = pl.BlockSpec((BR, CW), lambda i: (i, 0))
    full = lambda shape: pl.BlockSpec(shape, lambda i: (0, 0))
    outs = pl.pallas_call(
        body,
        grid=(GRID,),
        in_specs=[cspec] * (2 * H) + [
            full((1, din)),
            full((din, GIN_DIM)),
            full((1, GIN_DIM)),
            full((1, GIN_DIM)),
            full((1, GIN_DIM)),
            full((GIN_DIM, GIN_DIM)),
            full((1, GIN_DIM)),
        ],
        out_specs=[cspec, cspec, full((8, GIN_DIM))],
        out_shape=[
            jax.ShapeDtypeStruct((NPAD, CW), jnp.float32),
            jax.ShapeDtypeStruct((NPAD, CW), jnp.float32),
            jax.ShapeDtypeStruct((8, GIN_DIM), jnp.float32),
        ],
    )(*node_chunks, *agg_chunks, se,
      Wa, ba.reshape(1, -1), g.reshape(1, -1), be.reshape(1, -1),
      Wb, bb.reshape(1, -1))
    return outs[0], outs[1], outs[2]


def _classifier(g8, Wc1, bc1, Wc2p, bc2p):
    def body(g_r, w1_r, b1_r, w2_r, b2_r, o_r):
        h = jnp.dot(g_r[...], w1_r[...], preferred_element_type=jnp.float32)
        h = jnp.maximum(h + b1_r[...], 0.0)
        o_r[...] = jnp.dot(h, w2_r[...],
                           preferred_element_type=jnp.float32) + b2_r[...]

    return pl.pallas_call(
        body,
        out_shape=jax.ShapeDtypeStruct((8, 128), jnp.float32),
    )(g8, Wc1, bc1.reshape(1, -1), Wc2p, bc2p)


def kernel(x, edge_index, emb, W1, b1, g1, be1, W2, b2,
           Wh1, bh1, gh1, beh1, Wh2, bh2, eps1, eps2, eps3,
           Wc1, bc1, Wc2, bc2):
    xp = jnp.pad(x.reshape(-1), (0, NPAD - N)).reshape(XROWS, 128)
    src = jnp.pad(edge_index[0], (0, EPAD - E)).reshape(EROWS, 128)
    dst = jnp.pad(edge_index[1], (0, EPAD - E),
                  constant_values=TRASH).reshape(EROWS, 128)
    zeros = jnp.zeros((TROWS, CW), jnp.float32)
    emb_chunks = [emb[:, CW * h:CW * (h + 1)] for h in range(EMB_DIM // CW)]

    n0 = _emb_gather(xp, emb_chunks)
    a1 = _segment_sum(src, dst, zeros, n0)
    se1 = (1.0 + eps1) * jnp.ones((1, EMB_DIM), jnp.float32)
    n1a, n1b, p1 = _mlp(n0, a1, se1, W1, b1, g1, be1, W2, b2)

    se_h = jnp.ones((1, GIN_DIM), jnp.float32)
    a2 = _segment_sum(src, dst, zeros, [n1a, n1b])
    n2a, n2b, p2 = _mlp([n1a, n1b], a2, (1.0 + eps2) * se_h,
                        Wh1, bh1, gh1, beh1, Wh2, bh2)

    a3 = _segment_sum(src, dst, zeros, [n2a, n2b])
    _, _, p3 = _mlp([n2a, n2b], a3, (1.0 + eps3) * se_h,
                    Wh1, bh1, gh1, beh1, Wh2, bh2)

    g8 = jnp.concatenate([p1, p2, p3], axis=1)
    Wc2p = jnp.pad(Wc2, ((0, 0), (0, 127)))
    bc2p = jnp.pad(bc2.reshape(1, 1), ((0, 0), (0, 127)))
    res = _classifier(g8, Wc1, bc1, Wc2p, bc2p)
    return res[0:1, 0:1]


# speedup vs baseline: 4.2287x; 4.2287x over previous
import functools

import jax
import jax.numpy as jnp
from jax import lax
from jax.experimental import pallas as pl
from jax.experimental.pallas import tpu as pltpu
from jax.experimental.pallas import tpu_sc as plsc

N = 50000
E = 800000
EMB_DIM = 128
GIN_DIM = 64
CW = 32
NPAD = 53248
EPAD = 802816
XROWS = NPAD // 128
EROWS = EPAD // 128
NC = 2
NS = 16
TPB = EROWS // NS
KB = 4
ACC_ROWS = 50016
TRASH = 50008
TROWS = ACC_ROWS // NS
BR = 1024
GRID = NPAD // BR

_mesh = plsc.VectorSubcoreMesh(core_axis_name="c", subcore_axis_name="s")
_sc_params = pltpu.CompilerParams(use_tc_tiling_on_sc=False,
                                  internal_scratch_in_bytes=65536)


def _emb_gather(x2d, emb_chunks):
    H = len(emb_chunks)
    wpw = XROWS // (NC * NS)

    @functools.partial(
        pl.kernel,
        out_type=[jax.ShapeDtypeStruct((NPAD, CW), jnp.float32)] * H,
        mesh=_mesh,
        scratch_types=[
            pltpu.VMEM((1, 128), jnp.int32),
            pltpu.VMEM((H, 128, CW), jnp.float32),
            pltpu.SemaphoreType.DMA,
        ],
        compiler_params=_sc_params,
    )
    def k(x_ref, *rest):
        embs = rest[:H]
        outs = rest[H:2 * H]
        xv, rows, sem = rest[2 * H:]
        c = lax.axis_index("c")
        s = lax.axis_index("s")
        wid = s * NC + c

        def body(i, carry):
            rb = wid * wpw + i
            pltpu.sync_copy(x_ref.at[pl.ds(rb, 1)], xv)
            descs = [pltpu.async_copy(embs[h].at[xv.at[0]], rows.at[h], sem)
                     for h in range(H)]
            for d in descs:
                d.wait()
            for h in range(H):
                pltpu.sync_copy(rows.at[h], outs[h].at[pl.ds(rb * 128, 128)])
            return carry

        lax.fori_loop(0, wpw, body, 0)

    return list(k(x2d, *emb_chunks))


def _segment_sum(src2d, dst2d, zeros, node_chunks):
    H = len(node_chunks)

    @functools.partial(
        pl.kernel,
        out_type=[jax.ShapeDtypeStruct((NPAD, CW), jnp.float32)] * H,
        mesh=_mesh,
        scratch_types=[
            pltpu.VMEM((KB, 128), jnp.int32),
            pltpu.VMEM((KB, 128), jnp.int32),
            pltpu.VMEM((KB, 128, CW), jnp.float32),
            pltpu.VMEM_SHARED((ACC_ROWS, CW), jnp.float32),
            pltpu.SemaphoreType.DMA,
        ],
        compiler_params=_sc_params,
    )
    def k(src_ref, dst_ref, z_ref, *rest):
        nodes = rest[:H]
        outs = rest[H:2 * H]
        sv, dv, rows, acc, sem = rest[2 * H:]
        c = lax.axis_index("c")
        s = lax.axis_index("s")
        for h in range(H):
            @pl.when(c == (h % NC))
            def _(h=h):
                node_h = nodes[h]
                out_h = outs[h]
                pltpu.sync_copy(z_ref, acc.at[pl.ds(s * TROWS, TROWS)])
                plsc.subcore_barrier()

                def body(i, carry):
                    rb = s * TPB + i * KB
                    pltpu.sync_copy(src_ref.at[pl.ds(rb, KB)], sv)
                    pltpu.sync_copy(dst_ref.at[pl.ds(rb, KB)], dv)
                    descs = [pltpu.async_copy(node_h.at[sv.at[j]],
                                              rows.at[j], sem)
                             for j in range(KB)]
                    for d in descs:
                        d.wait()
                    for j in range(KB):
                        pltpu.sync_copy(rows.at[j], acc.at[dv.at[j]],
                                        add=True)
                    return carry

                lax.fori_loop(0, TPB // KB, body, 0)
                plsc.subcore_barrier()
                pltpu.sync_copy(acc.at[pl.ds(s * TROWS, TROWS)],
                                out_h.at[pl.ds(s * TROWS, TROWS)])

    return list(k(src2d, dst2d, zeros, *node_chunks))


def _mlp(node_chunks, agg_chunks, se, Wa, ba, g, be, Wb, bb):
    H = len(node_chunks)
    din = H * CW

    def body(*refs):
        nrefs = refs[:H]
        arefs = refs[H:2 * H]
        se_r, wa_r, ba_r, g_r, be_r, wb_r, bb_r = refs[2 * H:2 * H + 7]
        o0, o1, pool = refs[2 * H + 7:]
        i = pl.program_id(0)
        x = jnp.concatenate([r[...] for r in nrefs], axis=1)
        a = jnp.concatenate([r[...] for r in arefs], axis=1)
        z = se_r[...] * x + a
        h1 = jnp.dot(z, wa_r[...], preferred_element_type=jnp.float32, precision=lax.Precision.HIGHEST) + ba_r[...]
        m = jnp.mean(h1, axis=1, keepdims=True)
        v = jnp.mean((h1 - m) ** 2, axis=1, keepdims=True)
        h1 = (h1 - m) * lax.rsqrt(v + 1e-5) * g_r[...] + be_r[...]
        h1 = jnp.maximum(h1, 0.0)
        y = jnp.dot(h1, wb_r[...], preferred_element_type=jnp.float32, precision=lax.Precision.HIGHEST) + bb_r[...]
        o0[...] = y[:, :CW]
        o1[...] = y[:, CW:]
        rows = i * BR + lax.broadcasted_iota(jnp.int32, (BR, 1), 0)
        part = jnp.sum(jnp.where(rows < N, y, 0.0), axis=0, keepdims=True)
        pb = jnp.broadcast_to(part, (8, GIN_DIM))

        @pl.when(i == 0)
        def _init():
            pool[...] = pb

        @pl.when(i != 0)
        def _acc():
            pool[...] += pb

    cspec = pl.BlockSpec((BR, CW), lambda i: (i, 0))
    full = lambda shape: pl.BlockSpec(shape, lambda i: (0, 0))
    outs = pl.pallas_call(
        body,
        grid=(GRID,),
        in_specs=[cspec] * (2 * H) + [
            full((1, din)),
            full((din, GIN_DIM)),
            full((1, GIN_DIM)),
            full((1, GIN_DIM)),
            full((1, GIN_DIM)),
            full((GIN_DIM, GIN_DIM)),
            full((1, GIN_DIM)),
        ],
        out_specs=[cspec, cspec, full((8, GIN_DIM))],
        out_shape=[
            jax.ShapeDtypeStruct((NPAD, CW), jnp.float32),
            jax.ShapeDtypeStruct((NPAD, CW), jnp.float32),
            jax.ShapeDtypeStruct((8, GIN_DIM), jnp.float32),
        ],
    )(*node_chunks, *agg_chunks, se,
      Wa, ba.reshape(1, -1), g.reshape(1, -1), be.reshape(1, -1),
      Wb, bb.reshape(1, -1))
    return outs[0], outs[1], outs[2]


def _classifier(g8, Wc1, bc1, Wc2p, bc2p):
    def body(g_r, w1_r, b1_r, w2_r, b2_r, o_r):
        h = jnp.dot(g_r[...], w1_r[...], preferred_element_type=jnp.float32, precision=lax.Precision.HIGHEST)
        h = jnp.maximum(h + b1_r[...], 0.0)
        o_r[...] = jnp.dot(h, w2_r[...],
                           preferred_element_type=jnp.float32, precision=lax.Precision.HIGHEST) + b2_r[...]

    return pl.pallas_call(
        body,
        out_shape=jax.ShapeDtypeStruct((8, 128), jnp.float32),
    )(g8, Wc1, bc1.reshape(1, -1), Wc2p, bc2p)


def kernel(x, edge_index, emb, W1, b1, g1, be1, W2, b2,
           Wh1, bh1, gh1, beh1, Wh2, bh2, eps1, eps2, eps3,
           Wc1, bc1, Wc2, bc2):
    xp = jnp.pad(x.reshape(-1), (0, NPAD - N)).reshape(XROWS, 128)
    src = jnp.pad(edge_index[0], (0, EPAD - E)).reshape(EROWS, 128)
    dst = jnp.pad(edge_index[1], (0, EPAD - E),
                  constant_values=TRASH).reshape(EROWS, 128)
    zeros = jnp.zeros((TROWS, CW), jnp.float32)
    emb_chunks = [emb[:, CW * h:CW * (h + 1)] for h in range(EMB_DIM // CW)]

    n0 = _emb_gather(xp, emb_chunks)
    a1 = _segment_sum(src, dst, zeros, n0)
    se1 = (1.0 + eps1) * jnp.ones((1, EMB_DIM), jnp.float32)
    n1a, n1b, p1 = _mlp(n0, a1, se1, W1, b1, g1, be1, W2, b2)

    se_h = jnp.ones((1, GIN_DIM), jnp.float32)
    a2 = _segment_sum(src, dst, zeros, [n1a, n1b])
    n2a, n2b, p2 = _mlp([n1a, n1b], a2, (1.0 + eps2) * se_h,
                        Wh1, bh1, gh1, beh1, Wh2, bh2)

    a3 = _segment_sum(src, dst, zeros, [n2a, n2b])
    _, _, p3 = _mlp([n2a, n2b], a3, (1.0 + eps3) * se_h,
                    Wh1, bh1, gh1, beh1, Wh2, bh2)

    g8 = jnp.concatenate([p1, p2, p3], axis=1)
    Wc2p = jnp.pad(Wc2, ((0, 0), (0, 127)))
    bc2p = jnp.pad(bc2.reshape(1, 1), ((0, 0), (0, 127)))
    res = _classifier(g8, Wc1, bc1, Wc2p, bc2p)
    return res[0:1, 0:1]
